# baseline (speedup 1.0000x reference)
import functools

import jax
import jax.numpy as jnp
import numpy as np
from jax.experimental import pallas as pl
from jax.experimental.pallas import tpu as pltpu

NUM_EXPERTS = 16
HIDDEN = 768
CAP_FACTORS = [1.5, 1.5, 1.5, 1.5, 1.0, 1.0, 1.0, 1.0, 1.0, 1.0, 1.0, 1.0, 0.5, 0.5, 0.5, 0.5]
BASE_CAP = 128
MAX_CAP = int(max(CAP_FACTORS) * BASE_CAP)


def _router_probs_kernel(x_ref, w_ref, b_ref, probs_ref, zsum_ref):
    g = pl.program_id(0)
    tb = pl.program_id(1)

    x = x_ref[0]
    w = w_ref[...]
    logits = jax.lax.dot_general(
        x, w, (((1,), (1,)), ((), ())), preferred_element_type=jnp.float32)
    logits = logits + b_ref[...]

    m = jnp.max(logits, axis=1, keepdims=True)
    e = jnp.exp(logits - m)
    s = jnp.sum(e, axis=1, keepdims=True)
    probs_ref[0] = (e / s).T

    logz = m + jnp.log(s)

    @pl.when(jnp.logical_and(g == 0, tb == 0))
    def _():
        zsum_ref[...] = jnp.zeros_like(zsum_ref)

    zsum_ref[...] += jnp.sum(logz * logz).reshape(1, 1)


def _topk_kernel(probs_ref, caps_ref, vals_ref, idx_ref, work_ref):
    R, T = work_ref.shape
    work_ref[...] = probs_ref[...]
    iota_t = jax.lax.broadcasted_iota(jnp.int32, (R, T), 1)
    iota_c = jax.lax.broadcasted_iota(jnp.int32, (R, MAX_CAP), 1)

    def body(c, _):
        cur = work_ref[...]
        m = jnp.max(cur, axis=1, keepdims=True)
        hit = cur == m
        idx = jnp.min(jnp.where(hit, iota_t, T), axis=1, keepdims=True)
        sel = iota_c == c
        vals_ref[...] = jnp.where(sel, m, vals_ref[...])
        idx_ref[...] = jnp.where(sel, idx, idx_ref[...])
        work_ref[...] = jnp.where(iota_t == idx, -jnp.inf, cur)
        return 0

    jax.lax.fori_loop(0, MAX_CAP, body, 0)

    caps = caps_ref[:, 0:1]
    live = iota_c < caps
    vals_ref[...] = jnp.where(live, vals_ref[...], 0.0)
    idx_ref[...] = jnp.where(live, idx_ref[...], -1)


def _materialize_kernel(idx_ref, vals_ref, disp_ref, comb_ref):
    tb = pl.program_id(1)
    Tb = disp_ref.shape[1]
    tid = tb * Tb + jax.lax.broadcasted_iota(jnp.int32, (Tb, 1), 0)
    idx = idx_ref[0]
    hit = idx == tid
    comb_ref[0] = jnp.where(hit, vals_ref[0], 0.0)
    disp_ref[0] = hit.astype(jnp.int8)


@functools.partial(jax.jit, static_argnames=())
def kernel(token_inputs, W, b, expert_capacity):
    x = token_inputs.astype(jnp.float32)
    G, T, H = x.shape
    E = NUM_EXPERTS
    C = MAX_CAP
    R = G * E

    Tb = 512
    nt = T // Tb
    probs_t, zsum = pl.pallas_call(
        _router_probs_kernel,
        grid=(G, nt),
        in_specs=[
            pl.BlockSpec((1, Tb, H), lambda g, t: (g, t, 0)),
            pl.BlockSpec((E, H), lambda g, t: (0, 0)),
            pl.BlockSpec((1, E), lambda g, t: (0, 0)),
        ],
        out_specs=[
            pl.BlockSpec((1, E, Tb), lambda g, t: (g, 0, t)),
            pl.BlockSpec((1, 1), lambda g, t: (0, 0)),
        ],
        out_shape=[
            jax.ShapeDtypeStruct((G, E, T), jnp.float32),
            jax.ShapeDtypeStruct((1, 1), jnp.float32),
        ],
    )(x, W, b.reshape(1, E))

    router_z_loss = (zsum[0, 0] / (G * T)).astype(jnp.float32)
    auxiliary_loss = jnp.zeros((), dtype=jnp.float32)

    factors = jnp.asarray(CAP_FACTORS, dtype=jnp.float32)
    caps = jnp.floor(factors * expert_capacity).astype(jnp.int32)
    caps_rows = jnp.broadcast_to(jnp.tile(caps, G)[:, None], (R, 128))

    probs_rows = probs_t.reshape(R, T)
    vals, idx = pl.pallas_call(
        _topk_kernel,
        in_specs=[
            pl.BlockSpec((R, T), lambda: (0, 0)),
            pl.BlockSpec((R, 128), lambda: (0, 0)),
        ],
        out_specs=[
            pl.BlockSpec((R, C), lambda: (0, 0)),
            pl.BlockSpec((R, C), lambda: (0, 0)),
        ],
        out_shape=[
            jax.ShapeDtypeStruct((R, C), jnp.float32),
            jax.ShapeDtypeStruct((R, C), jnp.int32),
        ],
        scratch_shapes=[pltpu.VMEM((R, T), jnp.float32)],
    )(probs_rows, caps_rows)

    idx3 = idx.reshape(G, 1, E * C)
    vals3 = vals.reshape(G, 1, E * C)
    Tb2 = 256
    nt2 = T // Tb2
    disp, comb = pl.pallas_call(
        _materialize_kernel,
        grid=(G, nt2),
        in_specs=[
            pl.BlockSpec((1, 1, E * C), lambda g, t: (g, 0, 0)),
            pl.BlockSpec((1, 1, E * C), lambda g, t: (g, 0, 0)),
        ],
        out_specs=[
            pl.BlockSpec((1, Tb2, E * C), lambda g, t: (g, t, 0)),
            pl.BlockSpec((1, Tb2, E * C), lambda g, t: (g, t, 0)),
        ],
        out_shape=[
            jax.ShapeDtypeStruct((G, T, E * C), jnp.int8),
            jax.ShapeDtypeStruct((G, T, E * C), jnp.float32),
        ],
    )(idx3, vals3)

    dispatch_mask = disp.reshape(G, T, E, C).astype(bool)
    combine_array = comb.reshape(G, T, E, C)
    return (dispatch_mask, combine_array, auxiliary_loss, router_z_loss)
